# baseline (speedup 1.0000x reference)
import jax, jax.numpy as jnp, math
from jax.experimental import pallas as pl


def _bdot(a, b):
    return jnp.dot(a.astype(jnp.bfloat16), b.astype(jnp.bfloat16),
                   preferred_element_type=jnp.float32)


def kernel(x, edge_index, edge_attr, batch, pat_idxs, Wl0, bl0, Wr0, Pr0, Po0, Pb0, Wl1, bl1, Wr1, Pr1, Po1, Pb1, Wl2, bl2, Wr2, Pr2, Po2, Pb2, We1, be1, We2, be2, Wg, bg, Wh, bh):
    x = x.at[:, :12].set(x[:, :12] / jnp.max(x[:, :12], axis=0, keepdims=True))
    src = edge_index[0]
    dst = edge_index[1]
    w = jnp.ones((src.shape[0],), dtype=jnp.float32)
    n = x.shape[0]
    convs = [(Wl0, bl0, Wr0), (Wl1, bl1, Wr1), (Wl2, bl2, Wr2)]
    pools = [(Pr0, Po0, Pb0), (Pr1, Po1, Pb1), (Pr2, Po2, Pb2)]
    xs = []
    for (Wl, bl, Wr), (Pr, Po, Pb) in zip(convs, pools):
        sums = jax.ops.segment_sum(x[src] * w[:, None], dst, num_segments=n)
        cnt = jax.ops.segment_sum(w, dst, num_segments=n)
        mean = jnp.where(cnt[:, None] > 0, sums / jnp.maximum(cnt, 1.0)[:, None], 0.0)
        x = jax.nn.relu(_bdot(mean, Wl) + bl + _bdot(x, Wr))
        agg = jax.ops.segment_sum(x[src] * w[:, None], dst, num_segments=n)
        score = jnp.tanh((_bdot(agg, Pr) + Pb + _bdot(x, Po)).reshape(-1))
        k = int(math.ceil(0.2 * n))
        perm = jnp.argsort(-score)[:k]
        x = x[perm] * score[perm][:, None]
        mask = jnp.zeros((n,), dtype=bool).at[perm].set(True)
        idx_map = jnp.zeros((n,), dtype=jnp.int32).at[perm].set(jnp.arange(k, dtype=jnp.int32))
        w = w * mask[src].astype(jnp.float32) * mask[dst].astype(jnp.float32)
        src = idx_map[src]
        dst = idx_map[dst]
        batch = batch[perm]
        n = k
        gmax = jax.ops.segment_max(x, batch, num_segments=1)
        gsum = jax.ops.segment_sum(x, batch, num_segments=1)
        gcnt = jax.ops.segment_sum(jnp.ones((n,), dtype=jnp.float32), batch, num_segments=1)
        gmean = gsum / jnp.maximum(gcnt, 1.0)[:, None]
        xs.append(jnp.concatenate([gmax, gmean], axis=1))
    h = jnp.sum(jnp.stack(xs), axis=0)
    h = jax.nn.relu(_bdot(h, We1) + be1)
    h = jax.nn.relu(_bdot(h, We2) + be2)
    grade = jax.nn.log_softmax(_bdot(h, Wg) + bg, axis=1)
    hazard = jax.nn.sigmoid(_bdot(h, Wh) + bh) * 6.0 - 3.0
    return (h, grade, hazard)


# speedup vs baseline: 1.0000x; 1.0000x over previous
import jax, jax.numpy as jnp, math
from jax.experimental import pallas as pl


def _bdot(a, b):
    return jnp.dot(a.astype(jnp.bfloat16), b.astype(jnp.bfloat16),
                   preferred_element_type=jnp.float32)


def _seg_sum_seq(vals, dst, n):
    acc = jnp.zeros((n, vals.shape[1]), dtype=jnp.float32)
    def body(e, a):
        return a.at[dst[e]].add(vals[e])
    return jax.lax.fori_loop(0, vals.shape[0], body, acc)

def kernel(x, edge_index, edge_attr, batch, pat_idxs, Wl0, bl0, Wr0, Pr0, Po0, Pb0, Wl1, bl1, Wr1, Pr1, Po1, Pb1, Wl2, bl2, Wr2, Pr2, Po2, Pb2, We1, be1, We2, be2, Wg, bg, Wh, bh):
    x = x.at[:, :12].set(x[:, :12] / jnp.max(x[:, :12], axis=0, keepdims=True))
    src = edge_index[0]
    dst = edge_index[1]
    w = jnp.ones((src.shape[0],), dtype=jnp.float32)
    n = x.shape[0]
    convs = [(Wl0, bl0, Wr0), (Wl1, bl1, Wr1), (Wl2, bl2, Wr2)]
    pools = [(Pr0, Po0, Pb0), (Pr1, Po1, Pb1), (Pr2, Po2, Pb2)]
    xs = []
    for (Wl, bl, Wr), (Pr, Po, Pb) in zip(convs, pools):
        sums = jax.ops.segment_sum(x[src] * w[:, None], dst, num_segments=n)
        cnt = jax.ops.segment_sum(w, dst, num_segments=n)
        mean = jnp.where(cnt[:, None] > 0, sums / jnp.maximum(cnt, 1.0)[:, None], 0.0)
        x = jax.nn.relu(_bdot(mean, Wl) + bl + _bdot(x, Wr))
        agg = jax.ops.segment_sum(x[src] * w[:, None], dst, num_segments=n)
        score = jnp.tanh((_bdot(agg, Pr) + Pb + _bdot(x, Po)).reshape(-1))
        k = int(math.ceil(0.2 * n))
        perm = jnp.argsort(-score)[:k]
        x = x[perm] * score[perm][:, None]
        mask = jnp.zeros((n,), dtype=bool).at[perm].set(True)
        idx_map = jnp.zeros((n,), dtype=jnp.int32).at[perm].set(jnp.arange(k, dtype=jnp.int32))
        w = w * mask[src].astype(jnp.float32) * mask[dst].astype(jnp.float32)
        src = idx_map[src]
        dst = idx_map[dst]
        batch = batch[perm]
        n = k
        gmax = jax.ops.segment_max(x, batch, num_segments=1)
        gsum = jax.ops.segment_sum(x, batch, num_segments=1)
        gcnt = jax.ops.segment_sum(jnp.ones((n,), dtype=jnp.float32), batch, num_segments=1)
        gmean = gsum / jnp.maximum(gcnt, 1.0)[:, None]
        xs.append(jnp.concatenate([gmax, gmean], axis=1))
    h = jnp.sum(jnp.stack(xs), axis=0)
    h = jax.nn.relu(_bdot(h, We1) + be1)
    h = jax.nn.relu(_bdot(h, We2) + be2)
    grade = jax.nn.log_softmax(_bdot(h, Wg) + bg, axis=1)
    hazard = jax.nn.sigmoid(_bdot(h, Wh) + bh) * 6.0 - 3.0
    return (h, grade, hazard)


# speedup vs baseline: 1.1050x; 1.1049x over previous
import jax, jax.numpy as jnp, math
from jax.experimental import pallas as pl


def _jdot(a, b):
    return jnp.dot(a.astype(jnp.bfloat16), b.astype(jnp.bfloat16),
                   preferred_element_type=jnp.float32)


def _mm_body(x_ref, w_ref, o_ref):
    o_ref[...] = jnp.dot(x_ref[...].astype(jnp.bfloat16),
                         w_ref[...].astype(jnp.bfloat16),
                         preferred_element_type=jnp.float32)


def _bdot(a, b):
    n, kk = a.shape
    m = b.shape[1]
    mp = max(m, 128)
    bp = b if m == mp else jnp.pad(b, ((0, 0), (0, mp - m)))
    bn = 1000 if n % 1000 == 0 else n
    if n < 8:
        ap = jnp.pad(a, ((0, 8 - n), (0, 0)))
        out = pl.pallas_call(
            _mm_body,
            in_specs=[pl.BlockSpec((8, kk), lambda: (0, 0)),
                      pl.BlockSpec((kk, mp), lambda: (0, 0))],
            out_specs=pl.BlockSpec((8, mp), lambda: (0, 0)),
            out_shape=jax.ShapeDtypeStruct((8, mp), jnp.float32),
        )(ap, bp)
        return out[:n, :m]
    out = pl.pallas_call(
        _mm_body,
        grid=(n // bn,),
        in_specs=[pl.BlockSpec((bn, kk), lambda i: (i, 0)),
                  pl.BlockSpec((kk, mp), lambda i: (0, 0))],
        out_specs=pl.BlockSpec((bn, mp), lambda i: (i, 0)),
        out_shape=jax.ShapeDtypeStruct((n, mp), jnp.float32),
    )(a, bp)
    return out[:, :m]


def _seg_sum_seq(vals, dst, n):
    acc = jnp.zeros((n, vals.shape[1]), dtype=jnp.float32)
    def body(e, a):
        return a.at[dst[e]].add(vals[e])
    return jax.lax.fori_loop(0, vals.shape[0], body, acc)

def kernel(x, edge_index, edge_attr, batch, pat_idxs, Wl0, bl0, Wr0, Pr0, Po0, Pb0, Wl1, bl1, Wr1, Pr1, Po1, Pb1, Wl2, bl2, Wr2, Pr2, Po2, Pb2, We1, be1, We2, be2, Wg, bg, Wh, bh):
    x = x.at[:, :12].set(x[:, :12] / jnp.max(x[:, :12], axis=0, keepdims=True))
    src = edge_index[0]
    dst = edge_index[1]
    w = jnp.ones((src.shape[0],), dtype=jnp.float32)
    n = x.shape[0]
    convs = [(Wl0, bl0, Wr0), (Wl1, bl1, Wr1), (Wl2, bl2, Wr2)]
    pools = [(Pr0, Po0, Pb0), (Pr1, Po1, Pb1), (Pr2, Po2, Pb2)]
    xs = []
    for (Wl, bl, Wr), (Pr, Po, Pb) in zip(convs, pools):
        sums = jax.ops.segment_sum(x[src] * w[:, None], dst, num_segments=n)
        cnt = jax.ops.segment_sum(w, dst, num_segments=n)
        mean = jnp.where(cnt[:, None] > 0, sums / jnp.maximum(cnt, 1.0)[:, None], 0.0)
        x = jax.nn.relu(_bdot(mean, Wl) + bl + _bdot(x, Wr))
        agg = jax.ops.segment_sum(x[src] * w[:, None], dst, num_segments=n)
        score = jnp.tanh((_bdot(agg, Pr) + Pb + _bdot(x, Po)).reshape(-1))
        k = int(math.ceil(0.2 * n))
        perm = jnp.argsort(-score)[:k]
        x = x[perm] * score[perm][:, None]
        mask = jnp.zeros((n,), dtype=bool).at[perm].set(True)
        idx_map = jnp.zeros((n,), dtype=jnp.int32).at[perm].set(jnp.arange(k, dtype=jnp.int32))
        w = w * mask[src].astype(jnp.float32) * mask[dst].astype(jnp.float32)
        src = idx_map[src]
        dst = idx_map[dst]
        batch = batch[perm]
        n = k
        gmax = jax.ops.segment_max(x, batch, num_segments=1)
        gsum = jax.ops.segment_sum(x, batch, num_segments=1)
        gcnt = jax.ops.segment_sum(jnp.ones((n,), dtype=jnp.float32), batch, num_segments=1)
        gmean = gsum / jnp.maximum(gcnt, 1.0)[:, None]
        xs.append(jnp.concatenate([gmax, gmean], axis=1))
    h = jnp.sum(jnp.stack(xs), axis=0)
    h = jax.nn.relu(_jdot(h, We1) + be1)
    h = jax.nn.relu(_jdot(h, We2) + be2)
    grade = jax.nn.log_softmax(_jdot(h, Wg) + bg, axis=1)
    hazard = jax.nn.sigmoid(_jdot(h, Wh) + bh) * 6.0 - 3.0
    return (h, grade, hazard)
